# Initial kernel scaffold; baseline (speedup 1.0000x reference)
import jax
import jax.numpy as jnp
from jax.experimental import pallas as pl


def kernel(lf_fea, W1, W2):
    raise NotImplementedError("write your pallas kernel here")


# speedup vs baseline: 5.7155x; 5.7155x over previous
import functools

import jax
import jax.numpy as jnp
from jax import lax
from jax.experimental import pallas as pl
from jax.experimental.pallas import tpu as pltpu
from jax.experimental.pallas import tpu_sc as plsc

_N, _U, _V, _C, _H, _W = 2, 5, 5, 64, 64, 64
_PNH, _PSH, _K = 8, 8, 6
_B = _N * _PNH
_P = _V * _W
_S = _U * _PSH
_D = _S * _C
_R = _B * _P * _K


def _k1_dist_topk(ysr_ref, yct_ref, out_ref):
    b = pl.program_id(0)
    x = ysr_ref[...]
    xt = yct_ref[...]
    g = jnp.dot(x, xt, preferred_element_type=jnp.float32)
    rn = jnp.sum(x * x, axis=1)
    d = rn[:, None] + rn[None, :] - 2.0 * g
    colio = lax.broadcasted_iota(jnp.int32, (_P, _P), 1)
    big = jnp.int32(1 << 30)
    rows = []
    for k in range(_K):
        minv = jnp.min(d, axis=1, keepdims=True)
        m = jnp.min(jnp.where(d <= minv, colio, big), axis=1)
        rows.append(b * (_K * _P) + k * _P + m)
        d = jnp.where(colio == m[:, None], jnp.float32(jnp.inf), d)
    rows.append(rows[0])
    rows.append(rows[0])
    out_ref[...] = jnp.stack(rows, axis=0)


def _k2_w1(yst_ref, w_ref, out_ref):
    y2 = yst_ref[...].reshape(_P * _S, _C)
    out_ref[...] = jnp.dot(y2, w_ref[...],
                           preferred_element_type=jnp.float32)


def _k35_ksum(sel_ref, out_ref):
    s = sel_ref[...]
    t = jnp.sum(s, axis=1)
    out_ref[...] = jnp.where(t >= 0, t, 0.1 * t)


def _k4_conv3(x_ref, w_ref, out_ref):
    nb = x_ref.shape[0]
    acc = jnp.zeros((nb * _U * _H, _C), jnp.float32)
    for du in range(3):
        for dh in range(3):
            xs = x_ref[:, du:du + _U, dh:dh + _H, :].reshape(nb * _U * _H,
                                                             2 * _C)
            acc = acc + jnp.dot(xs, w_ref[du, dh],
                                preferred_element_type=jnp.float32)
    acc = jnp.where(acc >= 0, acc, 0.1 * acc)
    out_ref[...] = acc.reshape(nb, _U, _H, _C)


def _sc_gather(table_f, gidx_f):
    info = plsc.get_sparse_core_info()
    nc, ns = info.num_cores, info.num_subcores
    nw = nc * ns
    rpw = _R // nw
    ch = 8
    for cand in (24, 16, 8):
        if rpw % cand == 0:
            ch = cand
            break
    nchunks = rpw // ch
    mesh = plsc.VectorSubcoreMesh(core_axis_name="c", subcore_axis_name="s")

    @functools.partial(
        pl.kernel, mesh=mesh,
        out_type=jax.ShapeDtypeStruct((_R, _D), jnp.float32),
        scratch_types=[
            pltpu.VMEM((ch,), jnp.int32),
            pltpu.VMEM((ch, _D), jnp.float32),
            pltpu.SemaphoreType.DMA,
        ],
    )
    def k(table_hbm, idx_hbm, out_hbm, idx_v, rows_v, sem):
        wid = lax.axis_index("s") * nc + lax.axis_index("c")

        def body(i, carry):
            base = wid * rpw + i * ch
            pltpu.sync_copy(idx_hbm.at[pl.ds(base, ch)], idx_v)
            pltpu.async_copy(table_hbm.at[idx_v], rows_v, sem).wait()
            pltpu.sync_copy(rows_v, out_hbm.at[pl.ds(base, ch)])
            return carry

        lax.fori_loop(0, nchunks, body, 0)

    return k(table_f, gidx_f)


def kernel(lf_fea, W1, W2):
    f32 = jnp.float32
    x7 = lf_fea.reshape(_N, _U, _V, _C, _PNH, _PSH, _W)
    yst = x7.transpose(0, 4, 2, 6, 1, 5, 3).reshape(_B, _P, _S, _C)
    ysr = yst.reshape(_B, _P, _D)
    yct = jnp.swapaxes(ysr, 1, 2)

    gid8 = pl.pallas_call(
        _k1_dist_topk,
        grid=(_B,),
        in_specs=[
            pl.BlockSpec((None, _P, _D), lambda b: (b, 0, 0)),
            pl.BlockSpec((None, _D, _P), lambda b: (b, 0, 0)),
        ],
        out_specs=pl.BlockSpec((None, 8, _P), lambda b: (b, 0, 0)),
        out_shape=jax.ShapeDtypeStruct((_B, 8, _P), jnp.int32),
    )(ysr, yct)
    gidx = gid8[:, :_K, :].transpose(0, 2, 1).reshape(_R)

    w1kt = W1.reshape(_C, _K, _C).transpose(1, 2, 0)
    t_tab = pl.pallas_call(
        _k2_w1,
        grid=(_B, _K),
        in_specs=[
            pl.BlockSpec((None, _P, _S, _C), lambda b, k: (b, 0, 0, 0)),
            pl.BlockSpec((None, _C, _C), lambda b, k: (k, 0, 0)),
        ],
        out_specs=pl.BlockSpec((None, None, _P * _S, _C),
                               lambda b, k: (b, k, 0, 0)),
        out_shape=jax.ShapeDtypeStruct((_B, _K, _P * _S, _C), f32),
    )(yst, w1kt)
    table = t_tab.reshape(_R, _D)

    sel = _sc_gather(table, gidx)

    out1 = pl.pallas_call(
        _k35_ksum,
        grid=(_B * _P // 64,),
        in_specs=[pl.BlockSpec((64, _K, _D), lambda i: (i, 0, 0))],
        out_specs=pl.BlockSpec((64, _D), lambda i: (i, 0)),
        out_shape=jax.ShapeDtypeStruct((_B * _P, _D), f32),
    )(sel.reshape(_B * _P, _K, _D))

    o7 = out1.reshape(_N, _PNH, _V, _W, _U, _PSH, _C)
    out1_cl = o7.transpose(0, 2, 3, 4, 1, 5, 6).reshape(_N * _V * _W,
                                                        _U, _H, _C)
    fv = lf_fea.reshape(_N, _U, _V, _C, _H, _W)
    fea_ver_cl = fv.transpose(0, 2, 5, 1, 4, 3).reshape(_N * _V * _W,
                                                        _U, _H, _C)
    cat = jnp.concatenate([fea_ver_cl, out1_cl], axis=-1)
    catp = jnp.pad(cat, ((0, 0), (1, 1), (1, 1), (0, 0)))
    w2t = W2.transpose(2, 3, 1, 0)

    nb_blk = 32
    k4out = pl.pallas_call(
        _k4_conv3,
        grid=(_N * _V * _W // nb_blk,),
        in_specs=[
            pl.BlockSpec((nb_blk, _U + 2, _H + 2, 2 * _C),
                         lambda i: (i, 0, 0, 0)),
            pl.BlockSpec((3, 3, 2 * _C, _C), lambda i: (0, 0, 0, 0)),
        ],
        out_specs=pl.BlockSpec((nb_blk, _U, _H, _C), lambda i: (i, 0, 0, 0)),
        out_shape=jax.ShapeDtypeStruct((_N * _V * _W, _U, _H, _C), f32),
    )(catp, w2t)

    out = k4out.reshape(_N, _V, _W, _U, _H, _C)
    out = out.transpose(0, 3, 1, 5, 4, 2).reshape(_N * _U * _V, _C, _H, _W)
    return out


# speedup vs baseline: 6.0457x; 1.0578x over previous
import functools

import jax
import jax.numpy as jnp
from jax import lax
from jax.experimental import pallas as pl
from jax.experimental.pallas import tpu as pltpu
from jax.experimental.pallas import tpu_sc as plsc

_N, _U, _V, _C, _H, _W = 2, 5, 5, 64, 64, 64
_PNH, _PSH, _K = 8, 8, 6
_B = _N * _PNH
_P = _V * _W
_S = _U * _PSH
_D = _S * _C
_R = _B * _P * _K


def _k1_dist_topk(ysr_ref, out_ref):
    b = pl.program_id(0)
    x = ysr_ref[...]
    g = lax.dot_general(x, x, (((1,), (1,)), ((), ())),
                        preferred_element_type=jnp.float32)
    rn = jnp.sum(x * x, axis=1)
    d = rn[:, None] + rn[None, :] - 2.0 * g
    colio = lax.broadcasted_iota(jnp.int32, (_P, _P), 1)
    big = jnp.int32(1 << 30)
    rows = []
    for k in range(_K):
        minv = jnp.min(d, axis=1, keepdims=True)
        m = jnp.min(jnp.where(d <= minv, colio, big), axis=1)
        rows.append(b * (_K * _P) + k * _P + m)
        d = jnp.where(colio == m[:, None], jnp.float32(jnp.inf), d)
    rows.append(rows[0])
    rows.append(rows[0])
    out_ref[...] = jnp.stack(rows, axis=0)


def _k2_w1(yst_ref, w_ref, out_ref):
    y2 = yst_ref[...].reshape(_P * _S, _C)
    out_ref[...] = jnp.dot(y2, w_ref[...],
                           preferred_element_type=jnp.float32)


def _k35_ksum(sel_ref, out_ref):
    s = sel_ref[...]
    t = jnp.sum(s, axis=1)
    t = jnp.where(t >= 0, t, 0.1 * t)
    out_ref[...] = t.reshape(_W, _U, _PSH, _C)


def _k4_conv3(xf_ref, xs_ref, wf_ref, ws_ref, out_ref):
    nb = xf_ref.shape[0]
    pad = ((0, 0), (1, 1), (1, 1), (0, 0))
    xf = jnp.pad(xf_ref[...], pad)
    xs = jnp.pad(xs_ref[...], pad)
    acc = jnp.zeros((nb * _U * _H, _C), jnp.float32)
    for du in range(3):
        for dh in range(3):
            a = xf[:, du:du + _U, dh:dh + _H, :].reshape(nb * _U * _H, _C)
            b = xs[:, du:du + _U, dh:dh + _H, :].reshape(nb * _U * _H, _C)
            acc = acc + jnp.dot(a, wf_ref[du, dh],
                                preferred_element_type=jnp.float32)
            acc = acc + jnp.dot(b, ws_ref[du, dh],
                                preferred_element_type=jnp.float32)
    acc = jnp.where(acc >= 0, acc, 0.1 * acc)
    out_ref[...] = acc.reshape(nb, _U, _H, _C)


def _sc_gather(table_f, gidx_f):
    info = plsc.get_sparse_core_info()
    nc, ns = info.num_cores, info.num_subcores
    nw = nc * ns
    rpw = _R // nw
    ch = 8
    for cand in (24, 16, 8):
        if rpw % cand == 0:
            ch = cand
            break
    nchunks = rpw // ch
    mesh = plsc.VectorSubcoreMesh(core_axis_name="c", subcore_axis_name="s")

    @functools.partial(
        pl.kernel, mesh=mesh,
        out_type=jax.ShapeDtypeStruct((_R, _D), jnp.float32),
        scratch_types=[
            pltpu.VMEM((ch,), jnp.int32),
            pltpu.VMEM((ch, _D), jnp.float32),
            pltpu.SemaphoreType.DMA,
        ],
    )
    def k(table_hbm, idx_hbm, out_hbm, idx_v, rows_v, sem):
        wid = lax.axis_index("s") * nc + lax.axis_index("c")

        def body(i, carry):
            base = wid * rpw + i * ch
            pltpu.sync_copy(idx_hbm.at[pl.ds(base, ch)], idx_v)
            pltpu.async_copy(table_hbm.at[idx_v], rows_v, sem).wait()
            pltpu.sync_copy(rows_v, out_hbm.at[pl.ds(base, ch)])
            return carry

        lax.fori_loop(0, nchunks, body, 0)

    return k(table_f, gidx_f)


def kernel(lf_fea, W1, W2):
    f32 = jnp.float32
    x7 = lf_fea.reshape(_N, _U, _V, _C, _PNH, _PSH, _W)
    yst = x7.transpose(0, 4, 2, 6, 1, 5, 3).reshape(_B, _P, _S, _C)
    ysr = yst.reshape(_B, _P, _D)

    gid8 = pl.pallas_call(
        _k1_dist_topk,
        grid=(_B,),
        in_specs=[
            pl.BlockSpec((None, _P, _D), lambda b: (b, 0, 0)),
        ],
        out_specs=pl.BlockSpec((None, 8, _P), lambda b: (b, 0, 0)),
        out_shape=jax.ShapeDtypeStruct((_B, 8, _P), jnp.int32),
    )(ysr)
    gidx = gid8[:, :_K, :].transpose(0, 2, 1).reshape(_R)

    w1kt = W1.reshape(_C, _K, _C).transpose(1, 2, 0)
    t_tab = pl.pallas_call(
        _k2_w1,
        grid=(_B, _K),
        in_specs=[
            pl.BlockSpec((None, _P, _S, _C), lambda b, k: (b, 0, 0, 0)),
            pl.BlockSpec((None, _C, _C), lambda b, k: (k, 0, 0)),
        ],
        out_specs=pl.BlockSpec((None, None, _P * _S, _C),
                               lambda b, k: (b, k, 0, 0)),
        out_shape=jax.ShapeDtypeStruct((_B, _K, _P * _S, _C), f32),
    )(yst, w1kt)
    table = t_tab.reshape(_R, _D)

    sel = _sc_gather(table, gidx)

    out7 = pl.pallas_call(
        _k35_ksum,
        grid=(_B * _P // _W,),
        in_specs=[pl.BlockSpec((_W, _K, _D), lambda i: (i, 0, 0))],
        out_specs=pl.BlockSpec(
            (None, None, _W, _U, None, _PSH, _C),
            lambda i: (i // 40, i % 5, 0, 0, (i // 5) % 8, 0, 0)),
        out_shape=jax.ShapeDtypeStruct(
            (_N, _V, _W, _U, _PNH, _PSH, _C), f32),
    )(sel.reshape(_B * _P, _K, _D))
    out1_cl = out7.reshape(_N * _V * _W, _U, _H, _C)

    fv = lf_fea.reshape(_N, _U, _V, _C, _H, _W)
    fea_ver_cl = fv.transpose(0, 2, 5, 1, 4, 3).reshape(_N * _V * _W,
                                                        _U, _H, _C)
    w2f = W2[:, :_C].transpose(2, 3, 1, 0)
    w2s = W2[:, _C:].transpose(2, 3, 1, 0)

    nb_blk = 16
    k4out = pl.pallas_call(
        _k4_conv3,
        grid=(_N * _V * _W // nb_blk,),
        in_specs=[
            pl.BlockSpec((nb_blk, _U, _H, _C), lambda i: (i, 0, 0, 0)),
            pl.BlockSpec((nb_blk, _U, _H, _C), lambda i: (i, 0, 0, 0)),
            pl.BlockSpec((3, 3, _C, _C), lambda i: (0, 0, 0, 0)),
            pl.BlockSpec((3, 3, _C, _C), lambda i: (0, 0, 0, 0)),
        ],
        out_specs=pl.BlockSpec((nb_blk, _U, _H, _C), lambda i: (i, 0, 0, 0)),
        out_shape=jax.ShapeDtypeStruct((_N * _V * _W, _U, _H, _C), f32),
    )(fea_ver_cl, out1_cl, w2f, w2s)

    out = k4out.reshape(_N, _V, _W, _U, _H, _C)
    out = out.transpose(0, 3, 1, 5, 4, 2).reshape(_N * _U * _V, _C, _H, _W)
    return out


# speedup vs baseline: 8.0925x; 1.3386x over previous
import functools

import jax
import jax.numpy as jnp
from jax import lax
from jax.experimental import pallas as pl
from jax.experimental.pallas import tpu as pltpu
from jax.experimental.pallas import tpu_sc as plsc

_N, _U, _V, _C, _H, _W = 2, 5, 5, 64, 64, 64
_PNH, _PSH, _K = 8, 8, 6
_B = _N * _PNH
_P = _V * _W
_S = _U * _PSH
_D = _S * _C
_R = _B * _P * _K


def _k1_dist_topk(ysr_ref, out_ref):
    b = pl.program_id(0)
    x = ysr_ref[...]
    g = lax.dot_general(x, x, (((1,), (1,)), ((), ())),
                        preferred_element_type=jnp.float32)
    rn = jnp.sum(x * x, axis=1)
    d = rn[:, None] + rn[None, :] - 2.0 * g
    colio = lax.broadcasted_iota(jnp.int32, (_P, _P), 1)
    big = jnp.int32(1 << 30)
    rows = []
    for k in range(_K):
        minv = jnp.min(d, axis=1, keepdims=True)
        m = jnp.min(jnp.where(d <= minv, colio, big), axis=1)
        rows.append(b * _P + m)
        d = jnp.where(colio == m[:, None], jnp.float32(jnp.inf), d)
    rows.append(rows[0])
    rows.append(rows[0])
    out_ref[...] = jnp.stack(rows, axis=0)


def _k35_w1sum(sel_ref, w_ref, out_ref):
    acc = jnp.zeros((_W * _S, _C), jnp.float32)
    for k in range(_K):
        xk = sel_ref[:, k, :, :].reshape(_W * _S, _C)
        acc = acc + jnp.dot(xk, w_ref[k],
                            preferred_element_type=jnp.float32)
    acc = jnp.where(acc >= 0, acc, 0.1 * acc)
    out_ref[...] = acc.reshape(_W, _U, _PSH, _C)


def _k4_conv3(xf_ref, xs_ref, wf_ref, ws_ref, out_ref):
    nb = xf_ref.shape[0]
    pad = ((0, 0), (1, 1), (1, 1), (0, 0))
    xf = jnp.pad(xf_ref[...], pad)
    xs = jnp.pad(xs_ref[...], pad)
    acc = jnp.zeros((nb * _U * _H, _C), jnp.float32)
    for du in range(3):
        for dh in range(3):
            a = xf[:, du:du + _U, dh:dh + _H, :].reshape(nb * _U * _H, _C)
            b = xs[:, du:du + _U, dh:dh + _H, :].reshape(nb * _U * _H, _C)
            acc = acc + jnp.dot(a, wf_ref[du, dh],
                                preferred_element_type=jnp.float32)
            acc = acc + jnp.dot(b, ws_ref[du, dh],
                                preferred_element_type=jnp.float32)
    acc = jnp.where(acc >= 0, acc, 0.1 * acc)
    out_ref[...] = acc.reshape(nb, _U, _H, _C)


def _sc_gather(table_f, gidx_f):
    info = plsc.get_sparse_core_info()
    nc, ns = info.num_cores, info.num_subcores
    nw = nc * ns
    rpw = _R // nw
    ch = 8
    for cand in (24, 16, 8):
        if rpw % cand == 0:
            ch = cand
            break
    nchunks = rpw // ch
    mesh = plsc.VectorSubcoreMesh(core_axis_name="c", subcore_axis_name="s")

    @functools.partial(
        pl.kernel, mesh=mesh,
        out_type=jax.ShapeDtypeStruct((_R, _D), jnp.float32),
        scratch_types=[
            pltpu.VMEM((ch,), jnp.int32),
            pltpu.VMEM((ch, _D), jnp.float32),
            pltpu.SemaphoreType.DMA,
        ],
    )
    def k(table_hbm, idx_hbm, out_hbm, idx_v, rows_v, sem):
        wid = lax.axis_index("s") * nc + lax.axis_index("c")

        def body(i, carry):
            base = wid * rpw + i * ch
            pltpu.sync_copy(idx_hbm.at[pl.ds(base, ch)], idx_v)
            pltpu.async_copy(table_hbm.at[idx_v], rows_v, sem).wait()
            pltpu.sync_copy(rows_v, out_hbm.at[pl.ds(base, ch)])
            return carry

        lax.fori_loop(0, nchunks, body, 0)

    return k(table_f, gidx_f)


def kernel(lf_fea, W1, W2):
    f32 = jnp.float32
    x7 = lf_fea.reshape(_N, _U, _V, _C, _PNH, _PSH, _W)
    yst = x7.transpose(0, 4, 2, 6, 1, 5, 3).reshape(_B, _P, _S, _C)
    ysr = yst.reshape(_B, _P, _D)

    gid8 = pl.pallas_call(
        _k1_dist_topk,
        grid=(_B,),
        in_specs=[
            pl.BlockSpec((None, _P, _D), lambda b: (b, 0, 0)),
        ],
        out_specs=pl.BlockSpec((None, 8, _P), lambda b: (b, 0, 0)),
        out_shape=jax.ShapeDtypeStruct((_B, 8, _P), jnp.int32),
    )(ysr)
    gidx = gid8[:, :_K, :].transpose(0, 2, 1).reshape(_R)

    sel = _sc_gather(ysr.reshape(_B * _P, _D), gidx)

    w1kt = W1.reshape(_C, _K, _C).transpose(1, 2, 0)
    out7 = pl.pallas_call(
        _k35_w1sum,
        grid=(_B * _P // _W,),
        in_specs=[
            pl.BlockSpec((_W, _K, _S, _C), lambda i: (i, 0, 0, 0)),
            pl.BlockSpec((_K, _C, _C), lambda i: (0, 0, 0)),
        ],
        out_specs=pl.BlockSpec(
            (None, None, _W, _U, None, _PSH, _C),
            lambda i: (i // 40, i % 5, 0, 0, (i // 5) % 8, 0, 0)),
        out_shape=jax.ShapeDtypeStruct(
            (_N, _V, _W, _U, _PNH, _PSH, _C), f32),
    )(sel.reshape(_B * _P, _K, _S, _C), w1kt)
    out1_cl = out7.reshape(_N * _V * _W, _U, _H, _C)

    fv = lf_fea.reshape(_N, _U, _V, _C, _H, _W)
    fea_ver_cl = fv.transpose(0, 2, 5, 1, 4, 3).reshape(_N * _V * _W,
                                                        _U, _H, _C)
    w2f = W2[:, :_C].transpose(2, 3, 1, 0)
    w2s = W2[:, _C:].transpose(2, 3, 1, 0)

    nb_blk = 16
    k4out = pl.pallas_call(
        _k4_conv3,
        grid=(_N * _V * _W // nb_blk,),
        in_specs=[
            pl.BlockSpec((nb_blk, _U, _H, _C), lambda i: (i, 0, 0, 0)),
            pl.BlockSpec((nb_blk, _U, _H, _C), lambda i: (i, 0, 0, 0)),
            pl.BlockSpec((3, 3, _C, _C), lambda i: (0, 0, 0, 0)),
            pl.BlockSpec((3, 3, _C, _C), lambda i: (0, 0, 0, 0)),
        ],
        out_specs=pl.BlockSpec((nb_blk, _U, _H, _C), lambda i: (i, 0, 0, 0)),
        out_shape=jax.ShapeDtypeStruct((_N * _V * _W, _U, _H, _C), f32),
    )(fea_ver_cl, out1_cl, w2f, w2s)

    out = k4out.reshape(_N, _V, _W, _U, _H, _C)
    out = out.transpose(0, 3, 1, 5, 4, 2).reshape(_N * _U * _V, _C, _H, _W)
    return out


# speedup vs baseline: 8.9402x; 1.1048x over previous
import functools

import jax
import jax.numpy as jnp
from jax import lax
from jax.experimental import pallas as pl
from jax.experimental.pallas import tpu as pltpu
from jax.experimental.pallas import tpu_sc as plsc

_N, _U, _V, _C, _H, _W = 2, 5, 5, 64, 64, 64
_PNH, _PSH, _K = 8, 8, 6
_B = _N * _PNH
_P = _V * _W
_S = _U * _PSH
_D = _S * _C
_R = _B * _P * _K


def _k1_dist_topk(ysr_ref, out_ref):
    b = pl.program_id(0)
    x = ysr_ref[...]
    g = lax.dot_general(x, x, (((1,), (1,)), ((), ())),
                        preferred_element_type=jnp.float32)
    rn = jnp.sum(x * x, axis=1)
    d = rn[:, None] + rn[None, :] - 2.0 * g
    colio = lax.broadcasted_iota(jnp.int32, (_P, _P), 1)
    big = jnp.int32(1 << 30)
    rows = []
    for k in range(_K):
        minv = jnp.min(d, axis=1, keepdims=True)
        m = jnp.min(jnp.where(d <= minv, colio, big), axis=1)
        rows.append(b * _P + m)
        d = jnp.where(colio == m[:, None], jnp.float32(jnp.inf), d)
    rows.append(rows[0])
    rows.append(rows[0])
    out_ref[...] = jnp.stack(rows, axis=0)


def _k35_w1sum(sel_ref, w_ref, out_ref):
    acc = jnp.zeros((_W * _S, _C), jnp.float32)
    for k in range(_K):
        xk = sel_ref[:, k, :, :].reshape(_W * _S, _C)
        acc = acc + jnp.dot(xk, w_ref[k],
                            preferred_element_type=jnp.float32)
    acc = jnp.where(acc >= 0, acc, 0.1 * acc)
    out_ref[...] = acc.reshape(_W, _U, _PSH, _C)


def _k4_conv3(xf_ref, xs_ref, w_ref, out_ref):
    nb = xf_ref.shape[0]
    pad = ((0, 0), (1, 1), (1, 1), (0, 0))
    xc = jnp.concatenate([xf_ref[...], xs_ref[...]], axis=-1)
    xp = jnp.pad(xc, pad)
    acc = jnp.zeros((nb * _U * _H, _C), jnp.float32)
    for du in range(3):
        for dh in range(3):
            a = xp[:, du:du + _U, dh:dh + _H, :].reshape(nb * _U * _H,
                                                         2 * _C)
            acc = acc + jnp.dot(a, w_ref[du, dh],
                                preferred_element_type=jnp.float32)
    acc = jnp.where(acc >= 0, acc, 0.1 * acc)
    out_ref[...] = acc.reshape(nb, _U, _H, _C)


def _sc_gather(table_f, gidx_f):
    info = plsc.get_sparse_core_info()
    nc, ns = info.num_cores, info.num_subcores
    nw = nc * ns
    rpw = _R // nw
    ch = 8
    for cand in (16, 8):
        if rpw % (2 * cand) == 0:
            ch = cand
            break
    npairs = rpw // (2 * ch)
    mesh = plsc.VectorSubcoreMesh(core_axis_name="c", subcore_axis_name="s")

    @functools.partial(
        pl.kernel, mesh=mesh,
        out_type=jax.ShapeDtypeStruct((_R, _D), jnp.float32),
        scratch_types=[
            pltpu.VMEM((rpw,), jnp.int32),
            pltpu.VMEM((ch, _D), jnp.float32),
            pltpu.VMEM((ch, _D), jnp.float32),
            pltpu.SemaphoreType.DMA,
            pltpu.SemaphoreType.DMA,
            pltpu.SemaphoreType.DMA,
            pltpu.SemaphoreType.DMA,
        ],
    )
    def k(table_hbm, idx_hbm, out_hbm, idx_v, rows0, rows1,
          sg0, sg1, sw0, sw1):
        wid = lax.axis_index("s") * nc + lax.axis_index("c")
        wbase = wid * rpw
        pltpu.sync_copy(idx_hbm.at[pl.ds(wbase, rpw)], idx_v)

        def body(io, carry):
            for b, rows, sg, sw in ((0, rows0, sg0, sw0),
                                    (1, rows1, sg1, sw1)):
                j = 2 * io + b
                base = wbase + j * ch

                @pl.when(io > 0)
                def _():
                    pltpu.make_async_copy(
                        rows, out_hbm.at[pl.ds(base, ch)], sw).wait()

                pltpu.async_copy(
                    table_hbm.at[idx_v.at[pl.ds(j * ch, ch)]], rows, sg)
            for b, rows, sg, sw in ((0, rows0, sg0, sw0),
                                    (1, rows1, sg1, sw1)):
                j = 2 * io + b
                base = wbase + j * ch
                pltpu.make_async_copy(
                    table_hbm.at[idx_v.at[pl.ds(j * ch, ch)]],
                    rows, sg).wait()
                pltpu.async_copy(rows, out_hbm.at[pl.ds(base, ch)], sw)
            return carry

        lax.fori_loop(0, npairs, body, 0)
        for rows, sw in ((rows0, sw0), (rows1, sw1)):
            pltpu.make_async_copy(
                rows, out_hbm.at[pl.ds(wbase, ch)], sw).wait()

    return k(table_f, gidx_f)


def kernel(lf_fea, W1, W2):
    f32 = jnp.float32
    x7 = lf_fea.reshape(_N, _U, _V, _C, _PNH, _PSH, _W)
    yst = x7.transpose(0, 4, 2, 6, 1, 5, 3).reshape(_B, _P, _S, _C)
    ysr = yst.reshape(_B, _P, _D)

    gid8 = pl.pallas_call(
        _k1_dist_topk,
        grid=(_B,),
        in_specs=[
            pl.BlockSpec((None, _P, _D), lambda b: (b, 0, 0)),
        ],
        out_specs=pl.BlockSpec((None, 8, _P), lambda b: (b, 0, 0)),
        out_shape=jax.ShapeDtypeStruct((_B, 8, _P), jnp.int32),
    )(ysr)
    gidx = gid8[:, :_K, :].transpose(0, 2, 1).reshape(_R)

    sel = _sc_gather(ysr.reshape(_B * _P, _D), gidx)

    w1kt = W1.reshape(_C, _K, _C).transpose(1, 2, 0)
    out7 = pl.pallas_call(
        _k35_w1sum,
        grid=(_B * _P // _W,),
        in_specs=[
            pl.BlockSpec((_W, _K, _S, _C), lambda i: (i, 0, 0, 0)),
            pl.BlockSpec((_K, _C, _C), lambda i: (0, 0, 0)),
        ],
        out_specs=pl.BlockSpec(
            (None, None, _W, _U, None, _PSH, _C),
            lambda i: (i // 40, i % 5, 0, 0, (i // 5) % 8, 0, 0)),
        out_shape=jax.ShapeDtypeStruct(
            (_N, _V, _W, _U, _PNH, _PSH, _C), f32),
    )(sel.reshape(_B * _P, _K, _S, _C), w1kt)
    out1_cl = out7.reshape(_N * _V * _W, _U, _H, _C)

    fv = lf_fea.reshape(_N, _U, _V, _C, _H, _W)
    fea_ver_cl = fv.transpose(0, 2, 5, 1, 4, 3).reshape(_N * _V * _W,
                                                        _U, _H, _C)
    w2t = W2.transpose(2, 3, 1, 0)

    nb_blk = 16
    k4out = pl.pallas_call(
        _k4_conv3,
        grid=(_N * _V * _W // nb_blk,),
        in_specs=[
            pl.BlockSpec((nb_blk, _U, _H, _C), lambda i: (i, 0, 0, 0)),
            pl.BlockSpec((nb_blk, _U, _H, _C), lambda i: (i, 0, 0, 0)),
            pl.BlockSpec((3, 3, 2 * _C, _C), lambda i: (0, 0, 0, 0)),
        ],
        out_specs=pl.BlockSpec((nb_blk, _U, _H, _C), lambda i: (i, 0, 0, 0)),
        out_shape=jax.ShapeDtypeStruct((_N * _V * _W, _U, _H, _C), f32),
    )(fea_ver_cl, out1_cl, w2t)

    out = k4out.reshape(_N, _V, _W, _U, _H, _C)
    out = out.transpose(0, 3, 1, 5, 4, 2).reshape(_N * _U * _V, _C, _H, _W)
    return out


# speedup vs baseline: 8.9626x; 1.0025x over previous
import functools

import jax
import jax.numpy as jnp
from jax import lax
from jax.experimental import pallas as pl
from jax.experimental.pallas import tpu as pltpu
from jax.experimental.pallas import tpu_sc as plsc

_N, _U, _V, _C, _H, _W = 2, 5, 5, 64, 64, 64
_PNH, _PSH, _K = 8, 8, 6
_B = _N * _PNH
_P = _V * _W
_S = _U * _PSH
_D = _S * _C
_R = _B * _P * _K


def _k1_dist_topk(ysr_ref, out_ref):
    b = pl.program_id(0)
    x = ysr_ref[...]
    g = lax.dot_general(x, x, (((1,), (1,)), ((), ())),
                        preferred_element_type=jnp.float32)
    rn = jnp.sum(x * x, axis=1)
    d = rn[:, None] + rn[None, :] - 2.0 * g
    colio = lax.broadcasted_iota(jnp.int32, (_P, _P), 1)
    big = jnp.int32(1 << 30)
    rows = []
    for k in range(_K):
        minv = jnp.min(d, axis=1, keepdims=True)
        m = jnp.min(jnp.where(d <= minv, colio, big), axis=1)
        rows.append(b * _P + m)
        d = jnp.where(colio == m[:, None], jnp.float32(jnp.inf), d)
    rows.append(rows[0])
    rows.append(rows[0])
    out_ref[...] = jnp.stack(rows, axis=0)


def _k35_w1sum(sel_ref, w_ref, out_ref):
    acc = jnp.zeros((_W * _S, _C), jnp.float32)
    for k in range(_K):
        xk = sel_ref[:, k, :, :].reshape(_W * _S, _C)
        acc = acc + jnp.dot(xk, w_ref[k],
                            preferred_element_type=jnp.float32)
    acc = jnp.where(acc >= 0, acc, 0.1 * acc)
    out_ref[...] = acc.reshape(_W, _U, _PSH, _C)


def _k4_conv3(xf_ref, xs_ref, w_ref, out_ref):
    nb = xf_ref.shape[0]
    pad = ((0, 0), (1, 1), (1, 1), (0, 0))
    xc = jnp.concatenate([xf_ref[...], xs_ref[...]], axis=-1)
    xp = jnp.pad(xc, pad)
    acc = jnp.zeros((nb * _U * _H, _C), jnp.float32)
    for du in range(3):
        for dh in range(3):
            a = xp[:, du:du + _U, dh:dh + _H, :].reshape(nb * _U * _H,
                                                         2 * _C)
            acc = acc + jnp.dot(a, w_ref[du, dh],
                                preferred_element_type=jnp.float32)
    acc = jnp.where(acc >= 0, acc, 0.1 * acc)
    out_ref[...] = acc.reshape(nb, _U, _H, _C)


def _sc_gather(table_f, gidx_f):
    info = plsc.get_sparse_core_info()
    nc, ns = info.num_cores, info.num_subcores
    nw = nc * ns
    rpw = _R // nw
    ch = 8
    for cand in (24, 16, 8):
        if rpw % (2 * cand) == 0:
            ch = cand
            break
    npairs = rpw // (2 * ch)
    mesh = plsc.VectorSubcoreMesh(core_axis_name="c", subcore_axis_name="s")

    @functools.partial(
        pl.kernel, mesh=mesh,
        out_type=jax.ShapeDtypeStruct((_R, _D), jnp.float32),
        scratch_types=[
            pltpu.VMEM((rpw,), jnp.int32),
            pltpu.VMEM((ch, _D), jnp.float32),
            pltpu.VMEM((ch, _D), jnp.float32),
            pltpu.SemaphoreType.DMA,
            pltpu.SemaphoreType.DMA,
            pltpu.SemaphoreType.DMA,
            pltpu.SemaphoreType.DMA,
        ],
    )
    def k(table_hbm, idx_hbm, out_hbm, idx_v, rows0, rows1,
          sg0, sg1, sw0, sw1):
        wid = lax.axis_index("s") * nc + lax.axis_index("c")
        wbase = wid * rpw
        pltpu.sync_copy(idx_hbm.at[pl.ds(wbase, rpw)], idx_v)

        def body(io, carry):
            for b, rows, sg, sw in ((0, rows0, sg0, sw0),
                                    (1, rows1, sg1, sw1)):
                j = 2 * io + b
                base = wbase + j * ch

                @pl.when(io > 0)
                def _():
                    pltpu.make_async_copy(
                        rows, out_hbm.at[pl.ds(base, ch)], sw).wait()

                pltpu.async_copy(
                    table_hbm.at[idx_v.at[pl.ds(j * ch, ch)]], rows, sg)
            for b, rows, sg, sw in ((0, rows0, sg0, sw0),
                                    (1, rows1, sg1, sw1)):
                j = 2 * io + b
                base = wbase + j * ch
                pltpu.make_async_copy(
                    table_hbm.at[idx_v.at[pl.ds(j * ch, ch)]],
                    rows, sg).wait()
                pltpu.async_copy(rows, out_hbm.at[pl.ds(base, ch)], sw)
            return carry

        lax.fori_loop(0, npairs, body, 0)
        for rows, sw in ((rows0, sw0), (rows1, sw1)):
            pltpu.make_async_copy(
                rows, out_hbm.at[pl.ds(wbase, ch)], sw).wait()

    return k(table_f, gidx_f)


def kernel(lf_fea, W1, W2):
    f32 = jnp.float32
    x7 = lf_fea.reshape(_N, _U, _V, _C, _PNH, _PSH, _W)
    yst = x7.transpose(0, 4, 2, 6, 1, 5, 3).reshape(_B, _P, _S, _C)
    ysr = yst.reshape(_B, _P, _D)

    gid8 = pl.pallas_call(
        _k1_dist_topk,
        grid=(_B,),
        in_specs=[
            pl.BlockSpec((None, _P, _D), lambda b: (b, 0, 0)),
        ],
        out_specs=pl.BlockSpec((None, 8, _P), lambda b: (b, 0, 0)),
        out_shape=jax.ShapeDtypeStruct((_B, 8, _P), jnp.int32),
    )(ysr)
    gidx = gid8[:, :_K, :].transpose(0, 2, 1).reshape(_R)

    sel = _sc_gather(ysr.reshape(_B * _P, _D), gidx)

    w1kt = W1.reshape(_C, _K, _C).transpose(1, 2, 0)
    out7 = pl.pallas_call(
        _k35_w1sum,
        grid=(_B * _P // _W,),
        in_specs=[
            pl.BlockSpec((_W, _K, _S, _C), lambda i: (i, 0, 0, 0)),
            pl.BlockSpec((_K, _C, _C), lambda i: (0, 0, 0)),
        ],
        out_specs=pl.BlockSpec(
            (None, None, _W, _U, None, _PSH, _C),
            lambda i: (i // 40, i % 5, 0, 0, (i // 5) % 8, 0, 0)),
        out_shape=jax.ShapeDtypeStruct(
            (_N, _V, _W, _U, _PNH, _PSH, _C), f32),
    )(sel.reshape(_B * _P, _K, _S, _C), w1kt)
    out1_cl = out7.reshape(_N * _V * _W, _U, _H, _C)

    fv = lf_fea.reshape(_N, _U, _V, _C, _H, _W)
    fea_ver_cl = fv.transpose(0, 2, 5, 1, 4, 3).reshape(_N * _V * _W,
                                                        _U, _H, _C)
    w2t = W2.transpose(2, 3, 1, 0)

    nb_blk = 16
    k4out = pl.pallas_call(
        _k4_conv3,
        grid=(_N * _V * _W // nb_blk,),
        in_specs=[
            pl.BlockSpec((nb_blk, _U, _H, _C), lambda i: (i, 0, 0, 0)),
            pl.BlockSpec((nb_blk, _U, _H, _C), lambda i: (i, 0, 0, 0)),
            pl.BlockSpec((3, 3, 2 * _C, _C), lambda i: (0, 0, 0, 0)),
        ],
        out_specs=pl.BlockSpec((nb_blk, _U, _H, _C), lambda i: (i, 0, 0, 0)),
        out_shape=jax.ShapeDtypeStruct((_N * _V * _W, _U, _H, _C), f32),
    )(fea_ver_cl, out1_cl, w2t)

    out = k4out.reshape(_N, _V, _W, _U, _H, _C)
    out = out.transpose(0, 3, 1, 5, 4, 2).reshape(_N * _U * _V, _C, _H, _W)
    return out
